# baseline (speedup 1.0000x reference)
import functools

import jax
import jax.numpy as jnp
from jax import lax
from jax.experimental import pallas as pl
from jax.experimental.pallas import tpu as pltpu
from jax.experimental.pallas import tpu_sc as plsc

N = 100000
NC = 2
NS = 16
NW = NC * NS
CHUNK = 104
GRP = 8
NCHUNK = 16
B_PER_W = CHUNK * NCHUNK
NH = NW * B_PER_W
NPAD = 2 * NH
BN = 4096


def _sc_gather(emb_atom, atom_idx, batch_idx, charge_i):
    E1 = emb_atom.shape[1]
    mesh = plsc.VectorSubcoreMesh(core_axis_name="c", subcore_axis_name="s")

    @functools.partial(
        pl.kernel,
        out_type=(
            jax.ShapeDtypeStruct((NH, E1), jnp.float32),
            jax.ShapeDtypeStruct((NH,), jnp.int32),
        ),
        mesh=mesh,
        compiler_params=pltpu.CompilerParams(use_tc_tiling_on_sc=False),
        scratch_types=(
            [pltpu.VMEM((B_PER_W,), jnp.int32)] * 3 +
            [pltpu.VMEM((CHUNK, E1), jnp.float32)] * GRP +
            [pltpu.SemaphoreType.DMA] * (GRP + 2)
        ),
    )
    def k(table_hbm, idx_hbm, batch_hbm, charge_hbm, ea_hbm, cpn_hbm,
          idx_v, batch_v, cpn_v, *bufs_and_sems):
        rows = bufs_and_sems[:GRP]
        gsems = bufs_and_sems[GRP:2 * GRP]
        wsem, csem = bufs_and_sems[2 * GRP:]
        wid = lax.axis_index("s") * NC + lax.axis_index("c")
        base = wid * B_PER_W
        pltpu.sync_copy(idx_hbm.at[pl.ds(base, B_PER_W)], idx_v)
        pltpu.sync_copy(batch_hbm.at[pl.ds(base, B_PER_W)], batch_v)

        def body(i, carry):
            j0 = i * GRP
            cd = [pltpu.async_copy(
                charge_hbm.at[batch_v.at[pl.ds((j0 + k) * CHUNK, CHUNK)]],
                cpn_v.at[pl.ds((j0 + k) * CHUNK, CHUNK)], csem)
                for k in range(GRP)]
            gd = [pltpu.async_copy(
                table_hbm.at[idx_v.at[pl.ds((j0 + k) * CHUNK, CHUNK)]],
                rows[k], gsems[k])
                for k in range(GRP)]
            wd = []
            for k in range(GRP):
                gd[k].wait()
                wd.append(pltpu.async_copy(
                    rows[k], ea_hbm.at[pl.ds(base + (j0 + k) * CHUNK, CHUNK)],
                    wsem))
            for k in range(GRP):
                wd[k].wait()
                cd[k].wait()
            return carry

        lax.fori_loop(0, NCHUNK // GRP, body, 0)
        pltpu.sync_copy(cpn_v, cpn_hbm.at[pl.ds(base, B_PER_W)])

    return k(emb_atom, atom_idx, batch_idx, charge_i)


def _tc_fused(pos_feat, ea, cpn3, W1, b1r, W2, b2r, Wp_a, Wp_h, emb_charge,
              Wp_c, nb, blk_off, out_prev=None):
    IN = pos_feat.shape[1]
    E1 = ea.shape[1]
    VC, E3 = emb_charge.shape
    OUT = Wp_a.shape[1]

    def body(pf_ref, ea_ref, cpn_ref, w1_ref, b1_ref, w2_ref, b2_ref,
             wpa_ref, wph_ref, ec_ref, wpc_ref, *prev_and_out):
        out_ref = prev_and_out[-1]
        h1 = jnp.dot(pf_ref[...], w1_ref[...], preferred_element_type=jnp.float32)
        h1 = h1 + b1_ref[...]
        h1 = h1 * jax.nn.sigmoid(h1)
        h = jnp.dot(h1, w2_ref[...], preferred_element_type=jnp.float32) + b2_ref[...]
        acc = jnp.dot(ea_ref[...], wpa_ref[...], preferred_element_type=jnp.float32)
        acc = acc + jnp.dot(h, wph_ref[...], preferred_element_type=jnp.float32)
        cg = jnp.dot(ec_ref[...], wpc_ref[...], preferred_element_type=jnp.float32)
        cpn = cpn_ref[0, 0, :]
        oh = (cpn[:, None] == lax.broadcasted_iota(jnp.int32, (BN, VC), 1)
              ).astype(jnp.float32)
        acc = acc + jnp.dot(oh, cg, preferred_element_type=jnp.float32)
        out_ref[...] = acc * jax.nn.sigmoid(acc)

    rep = lambda i: (0, 0)
    in_specs = [
        pl.BlockSpec((BN, IN), lambda i: (i + blk_off, 0)),
        pl.BlockSpec((BN, E1), lambda i: (i, 0)),
        pl.BlockSpec((1, 1, BN), lambda i: (i, 0, 0)),
        pl.BlockSpec((IN, E1), rep),
        pl.BlockSpec((1, E1), rep),
        pl.BlockSpec((E1, E1), rep),
        pl.BlockSpec((1, E1), rep),
        pl.BlockSpec((E1, OUT), rep),
        pl.BlockSpec((E1, OUT), rep),
        pl.BlockSpec((VC, E3), rep),
        pl.BlockSpec((E3, OUT), rep),
    ]
    args = [pos_feat, ea, cpn3, W1, b1r, W2, b2r, Wp_a, Wp_h, emb_charge, Wp_c]
    aliases = {}
    if out_prev is not None:
        in_specs.append(pl.BlockSpec(memory_space=pl.ANY))
        args.append(out_prev)
        aliases = {11: 0}
    return pl.pallas_call(
        body,
        grid=(nb,),
        in_specs=in_specs,
        out_specs=pl.BlockSpec((BN, OUT), lambda i: (i + blk_off, 0)),
        out_shape=jax.ShapeDtypeStruct((N, OUT), jnp.float32),
        input_output_aliases=aliases,
    )(*args)


def kernel(batch, atom_type, pos_feat, charge, emb_atom, W1, b1, W2, b2, emb_charge, W_proj):
    E1 = emb_atom.shape[1]
    E2 = W2.shape[1]
    pad = NPAD - N
    atom_idx = jnp.pad(atom_type.astype(jnp.int32), (0, pad))
    batch_idx = jnp.pad(batch.astype(jnp.int32), (0, pad))
    charge_i = charge.astype(jnp.int32)
    Wp_a = W_proj[:E1]
    Wp_h = W_proj[E1:E1 + E2]
    Wp_c = W_proj[E1 + E2:]
    b1r, b2r = b1.reshape(1, -1), b2.reshape(1, -1)

    ea0, cpn0 = _sc_gather(emb_atom, atom_idx[:NH], batch_idx[:NH], charge_i)
    ea1, cpn1 = _sc_gather(emb_atom, atom_idx[NH:], batch_idx[NH:], charge_i)
    cpn3_0 = cpn0.reshape(NH // BN, 1, BN)
    cpn3_1 = cpn1.reshape(NH // BN, 1, BN)

    nb0 = NH // BN
    nb1 = pl.cdiv(N - NH, BN)
    out0 = _tc_fused(pos_feat, ea0, cpn3_0, W1, b1r, W2, b2r, Wp_a, Wp_h,
                     emb_charge, Wp_c, nb0, 0)
    return _tc_fused(pos_feat, ea1, cpn3_1, W1, b1r, W2, b2r, Wp_a, Wp_h,
                     emb_charge, Wp_c, nb1, nb0, out0)


# speedup vs baseline: 1.5063x; 1.5063x over previous
import functools

import jax
import jax.numpy as jnp
from jax import lax
from jax.experimental import pallas as pl
from jax.experimental.pallas import tpu as pltpu
from jax.experimental.pallas import tpu_sc as plsc

N = 100000
NC = 2
NS = 16
NW = NC * NS
CHUNK = 128
GRP = 5
NCHUNK = 25
B_PER_W = CHUNK * NCHUNK
NPAD = NW * B_PER_W
BN = 4096


TAIL_W = NW - 1
TAIL_ROWS = N - TAIL_W * B_PER_W
TAIL_FULL = TAIL_ROWS // CHUNK
TAIL_REM = TAIL_ROWS - TAIL_FULL * CHUNK


def _sc_gather(emb_atom, atom_idx, batch_idx, charge_i):
    E1 = emb_atom.shape[1]
    mesh = plsc.VectorSubcoreMesh(core_axis_name="c", subcore_axis_name="s")

    @functools.partial(
        pl.kernel,
        out_type=(
            jax.ShapeDtypeStruct((N, E1), jnp.float32),
            jax.ShapeDtypeStruct((NPAD,), jnp.int32),
        ),
        mesh=mesh,
        compiler_params=pltpu.CompilerParams(use_tc_tiling_on_sc=False),
        scratch_types=(
            [pltpu.VMEM((B_PER_W,), jnp.int32)] * 3 +
            [pltpu.VMEM((CHUNK, E1), jnp.float32)] * GRP +
            [pltpu.SemaphoreType.DMA] * (GRP + 2)
        ),
    )
    def k(table_hbm, idx_hbm, batch_hbm, charge_hbm, ea_hbm, cpn_hbm,
          idx_v, batch_v, cpn_v, *bufs_and_sems):
        rows = bufs_and_sems[:GRP]
        gsems = bufs_and_sems[GRP:2 * GRP]
        wsem, csem = bufs_and_sems[2 * GRP:]
        wid = lax.axis_index("s") * NC + lax.axis_index("c")
        base = wid * B_PER_W

        @pl.when(wid < TAIL_W)
        def _main():
            pltpu.sync_copy(idx_hbm.at[pl.ds(base, B_PER_W)], idx_v)
            pltpu.sync_copy(batch_hbm.at[pl.ds(base, B_PER_W)], batch_v)

            def body(i, carry):
                j0 = i * GRP
                cd = [pltpu.async_copy(
                    charge_hbm.at[batch_v.at[pl.ds((j0 + k) * CHUNK, CHUNK)]],
                    cpn_v.at[pl.ds((j0 + k) * CHUNK, CHUNK)], csem)
                    for k in range(GRP)]
                gd = [pltpu.async_copy(
                    table_hbm.at[idx_v.at[pl.ds((j0 + k) * CHUNK, CHUNK)]],
                    rows[k], gsems[k])
                    for k in range(GRP)]
                wd = []
                for k in range(GRP):
                    gd[k].wait()
                    wd.append(pltpu.async_copy(
                        rows[k],
                        ea_hbm.at[pl.ds(base + (j0 + k) * CHUNK, CHUNK)],
                        wsem))
                for k in range(GRP):
                    wd[k].wait()
                    cd[k].wait()
                return carry

            lax.fori_loop(0, NCHUNK // GRP, body, 0)
            pltpu.sync_copy(cpn_v, cpn_hbm.at[pl.ds(base, B_PER_W)])

        @pl.when(wid == TAIL_W)
        def _tail():
            tb = TAIL_W * B_PER_W
            pltpu.sync_copy(idx_hbm.at[pl.ds(tb, TAIL_ROWS)],
                            idx_v.at[pl.ds(0, TAIL_ROWS)])
            pltpu.sync_copy(batch_hbm.at[pl.ds(tb, TAIL_ROWS)],
                            batch_v.at[pl.ds(0, TAIL_ROWS)])
            sizes = [CHUNK] * TAIL_FULL + ([TAIL_REM] if TAIL_REM else [])
            cd = []
            off = 0
            for sz in sizes:
                cd.append(pltpu.async_copy(
                    charge_hbm.at[batch_v.at[pl.ds(off, sz)]],
                    cpn_v.at[pl.ds(off, sz)], csem))
                off += sz
            off = 0
            for j, sz in enumerate(sizes):
                buf = rows[j % GRP] if sz == CHUNK else rows[j % GRP].at[pl.ds(0, sz)]
                pltpu.async_copy(
                    table_hbm.at[idx_v.at[pl.ds(off, sz)]], buf,
                    gsems[j % GRP]).wait()
                pltpu.sync_copy(buf, ea_hbm.at[pl.ds(tb + off, sz)])
                off += sz
            for c in cd:
                c.wait()
            pltpu.sync_copy(cpn_v.at[pl.ds(0, TAIL_ROWS)],
                            cpn_hbm.at[pl.ds(tb, TAIL_ROWS)])

    return k(emb_atom, atom_idx, batch_idx, charge_i)


def _tc_fused(pos_feat, ea, cpn3, W1, b1r, W2, b2r, Wp_a, Wp_h, emb_charge, Wp_c):
    IN = pos_feat.shape[1]
    E1 = ea.shape[1]
    VC, E3 = emb_charge.shape
    OUT = Wp_a.shape[1]
    nb = pl.cdiv(N, BN)

    def body(pf_ref, ea_ref, cpn_ref, w1_ref, b1_ref, w2_ref, b2_ref,
             wpa_ref, wph_ref, ec_ref, wpc_ref, out_ref):
        h1 = jnp.dot(pf_ref[...], w1_ref[...], preferred_element_type=jnp.float32)
        h1 = h1 + b1_ref[...]
        h1 = h1 * jax.nn.sigmoid(h1)
        h = jnp.dot(h1, w2_ref[...], preferred_element_type=jnp.float32) + b2_ref[...]
        acc = jnp.dot(ea_ref[...], wpa_ref[...], preferred_element_type=jnp.float32)
        acc = acc + jnp.dot(h, wph_ref[...], preferred_element_type=jnp.float32)
        cg = jnp.dot(ec_ref[...], wpc_ref[...], preferred_element_type=jnp.float32)
        cpn = cpn_ref[0, 0, :]
        oh = (cpn[:, None] == lax.broadcasted_iota(jnp.int32, (BN, VC), 1)
              ).astype(jnp.float32)
        acc = acc + jnp.dot(oh, cg, preferred_element_type=jnp.float32)
        out_ref[...] = acc * jax.nn.sigmoid(acc)

    rep = lambda i: (0, 0)
    return pl.pallas_call(
        body,
        grid=(nb,),
        in_specs=[
            pl.BlockSpec((BN, IN), lambda i: (i, 0)),
            pl.BlockSpec((BN, E1), lambda i: (i, 0)),
            pl.BlockSpec((1, 1, BN), lambda i: (i, 0, 0)),
            pl.BlockSpec((IN, E1), rep),
            pl.BlockSpec((1, E1), rep),
            pl.BlockSpec((E1, E1), rep),
            pl.BlockSpec((1, E1), rep),
            pl.BlockSpec((E1, OUT), rep),
            pl.BlockSpec((E1, OUT), rep),
            pl.BlockSpec((VC, E3), rep),
            pl.BlockSpec((E3, OUT), rep),
        ],
        out_specs=pl.BlockSpec((BN, OUT), lambda i: (i, 0)),
        out_shape=jax.ShapeDtypeStruct((N, OUT), jnp.float32),
    )(pos_feat, ea, cpn3, W1, b1r, W2, b2r, Wp_a, Wp_h, emb_charge, Wp_c)


def kernel(batch, atom_type, pos_feat, charge, emb_atom, W1, b1, W2, b2, emb_charge, W_proj):
    E1 = emb_atom.shape[1]
    E2 = W2.shape[1]
    atom_idx = atom_type.astype(jnp.int32)
    batch_idx = batch.astype(jnp.int32)
    charge_i = charge.astype(jnp.int32)
    Wp_a = W_proj[:E1]
    Wp_h = W_proj[E1:E1 + E2]
    Wp_c = W_proj[E1 + E2:]
    b1r, b2r = b1.reshape(1, -1), b2.reshape(1, -1)

    ea, cpn = _sc_gather(emb_atom, atom_idx, batch_idx, charge_i)
    cpn3 = cpn.reshape(NPAD // BN, 1, BN)
    return _tc_fused(pos_feat, ea, cpn3, W1, b1r, W2, b2r, Wp_a, Wp_h,
                     emb_charge, Wp_c)


# speedup vs baseline: 1.5305x; 1.0160x over previous
import functools

import jax
import jax.numpy as jnp
from jax import lax
from jax.experimental import pallas as pl
from jax.experimental.pallas import tpu as pltpu
from jax.experimental.pallas import tpu_sc as plsc

N = 100000
NC = 2
NS = 16
NW = NC * NS
CHUNK = 128
GRP = 12
NFULL = 24
NCHUNK = 25
B_PER_W = CHUNK * NCHUNK
NPAD = NW * B_PER_W
BN = 8192
CPAD = -(-N // BN) * BN


TAIL_W = NW - 1
TAIL_ROWS = N - TAIL_W * B_PER_W
TAIL_FULL = TAIL_ROWS // CHUNK
TAIL_REM = TAIL_ROWS - TAIL_FULL * CHUNK


def _sc_gather(emb_atom, atom_idx, batch_idx, charge_i):
    E1 = emb_atom.shape[1]
    mesh = plsc.VectorSubcoreMesh(core_axis_name="c", subcore_axis_name="s")

    @functools.partial(
        pl.kernel,
        out_type=(
            jax.ShapeDtypeStruct((N, E1), jnp.float32),
            jax.ShapeDtypeStruct((CPAD,), jnp.int32),
        ),
        mesh=mesh,
        compiler_params=pltpu.CompilerParams(use_tc_tiling_on_sc=False),
        scratch_types=(
            [pltpu.VMEM((B_PER_W,), jnp.int32)] * 3 +
            [pltpu.VMEM((CHUNK, E1), jnp.float32)] * GRP +
            [pltpu.SemaphoreType.DMA] * (GRP + 2)
        ),
    )
    def k(table_hbm, idx_hbm, batch_hbm, charge_hbm, ea_hbm, cpn_hbm,
          idx_v, batch_v, cpn_v, *bufs_and_sems):
        rows = bufs_and_sems[:GRP]
        gsems = bufs_and_sems[GRP:2 * GRP]
        wsem, csem = bufs_and_sems[2 * GRP:]
        wid = lax.axis_index("s") * NC + lax.axis_index("c")
        base = wid * B_PER_W

        @pl.when(wid < TAIL_W)
        def _main():
            pltpu.sync_copy(idx_hbm.at[pl.ds(base, B_PER_W)], idx_v)
            pltpu.sync_copy(batch_hbm.at[pl.ds(base, B_PER_W)], batch_v)

            def body(i, carry):
                j0 = i * GRP
                cd = [pltpu.async_copy(
                    charge_hbm.at[batch_v.at[pl.ds((j0 + k) * CHUNK, CHUNK)]],
                    cpn_v.at[pl.ds((j0 + k) * CHUNK, CHUNK)], csem)
                    for k in range(GRP)]
                gd = [pltpu.async_copy(
                    table_hbm.at[idx_v.at[pl.ds((j0 + k) * CHUNK, CHUNK)]],
                    rows[k], gsems[k])
                    for k in range(GRP)]
                wd = []
                for k in range(GRP):
                    gd[k].wait()
                    wd.append(pltpu.async_copy(
                        rows[k],
                        ea_hbm.at[pl.ds(base + (j0 + k) * CHUNK, CHUNK)],
                        wsem))
                for k in range(GRP):
                    wd[k].wait()
                    cd[k].wait()
                return carry

            lax.fori_loop(0, NFULL // GRP, body, 0)
            last = NCHUNK - 1
            cl = pltpu.async_copy(
                charge_hbm.at[batch_v.at[pl.ds(last * CHUNK, CHUNK)]],
                cpn_v.at[pl.ds(last * CHUNK, CHUNK)], csem)
            pltpu.async_copy(
                table_hbm.at[idx_v.at[pl.ds(last * CHUNK, CHUNK)]],
                rows[0], gsems[0]).wait()
            pltpu.sync_copy(rows[0], ea_hbm.at[pl.ds(base + last * CHUNK, CHUNK)])
            cl.wait()
            pltpu.sync_copy(cpn_v, cpn_hbm.at[pl.ds(base, B_PER_W)])

        @pl.when(wid == TAIL_W)
        def _tail():
            tb = TAIL_W * B_PER_W
            pltpu.sync_copy(idx_hbm.at[pl.ds(tb, TAIL_ROWS)],
                            idx_v.at[pl.ds(0, TAIL_ROWS)])
            pltpu.sync_copy(batch_hbm.at[pl.ds(tb, TAIL_ROWS)],
                            batch_v.at[pl.ds(0, TAIL_ROWS)])
            sizes = [CHUNK] * TAIL_FULL + ([TAIL_REM] if TAIL_REM else [])
            cd = []
            off = 0
            for sz in sizes:
                cd.append(pltpu.async_copy(
                    charge_hbm.at[batch_v.at[pl.ds(off, sz)]],
                    cpn_v.at[pl.ds(off, sz)], csem))
                off += sz
            off = 0
            for j, sz in enumerate(sizes):
                buf = rows[j % GRP] if sz == CHUNK else rows[j % GRP].at[pl.ds(0, sz)]
                pltpu.async_copy(
                    table_hbm.at[idx_v.at[pl.ds(off, sz)]], buf,
                    gsems[j % GRP]).wait()
                pltpu.sync_copy(buf, ea_hbm.at[pl.ds(tb + off, sz)])
                off += sz
            for c in cd:
                c.wait()
            pltpu.sync_copy(cpn_v.at[pl.ds(0, TAIL_ROWS)],
                            cpn_hbm.at[pl.ds(tb, TAIL_ROWS)])

    return k(emb_atom, atom_idx, batch_idx, charge_i)


def _tc_fused(pos_feat, ea, cpn3, W1, b1r, W2, b2r, Wp_a, Wp_h, emb_charge, Wp_c):
    IN = pos_feat.shape[1]
    E1 = ea.shape[1]
    VC, E3 = emb_charge.shape
    OUT = Wp_a.shape[1]
    nb = pl.cdiv(N, BN)

    def body(pf_ref, ea_ref, cpn_ref, w1_ref, b1_ref, w2_ref, b2_ref,
             wpa_ref, wph_ref, ec_ref, wpc_ref, out_ref):
        h1 = jnp.dot(pf_ref[...], w1_ref[...], preferred_element_type=jnp.float32)
        h1 = h1 + b1_ref[...]
        h1 = h1 * jax.nn.sigmoid(h1)
        h = jnp.dot(h1, w2_ref[...], preferred_element_type=jnp.float32) + b2_ref[...]
        acc = jnp.dot(ea_ref[...], wpa_ref[...], preferred_element_type=jnp.float32)
        acc = acc + jnp.dot(h, wph_ref[...], preferred_element_type=jnp.float32)
        cg = jnp.dot(ec_ref[...], wpc_ref[...], preferred_element_type=jnp.float32)
        cpn = cpn_ref[0, 0, :]
        oh = (cpn[:, None] == lax.broadcasted_iota(jnp.int32, (BN, VC), 1)
              ).astype(jnp.float32)
        acc = acc + jnp.dot(oh, cg, preferred_element_type=jnp.float32)
        out_ref[...] = acc * jax.nn.sigmoid(acc)

    rep = lambda i: (0, 0)
    return pl.pallas_call(
        body,
        grid=(nb,),
        in_specs=[
            pl.BlockSpec((BN, IN), lambda i: (i, 0)),
            pl.BlockSpec((BN, E1), lambda i: (i, 0)),
            pl.BlockSpec((1, 1, BN), lambda i: (i, 0, 0)),
            pl.BlockSpec((IN, E1), rep),
            pl.BlockSpec((1, E1), rep),
            pl.BlockSpec((E1, E1), rep),
            pl.BlockSpec((1, E1), rep),
            pl.BlockSpec((E1, OUT), rep),
            pl.BlockSpec((E1, OUT), rep),
            pl.BlockSpec((VC, E3), rep),
            pl.BlockSpec((E3, OUT), rep),
        ],
        out_specs=pl.BlockSpec((BN, OUT), lambda i: (i, 0)),
        out_shape=jax.ShapeDtypeStruct((N, OUT), jnp.float32),
    )(pos_feat, ea, cpn3, W1, b1r, W2, b2r, Wp_a, Wp_h, emb_charge, Wp_c)


def kernel(batch, atom_type, pos_feat, charge, emb_atom, W1, b1, W2, b2, emb_charge, W_proj):
    E1 = emb_atom.shape[1]
    E2 = W2.shape[1]
    atom_idx = atom_type.astype(jnp.int32)
    batch_idx = batch.astype(jnp.int32)
    charge_i = charge.astype(jnp.int32)
    Wp_a = W_proj[:E1]
    Wp_h = W_proj[E1:E1 + E2]
    Wp_c = W_proj[E1 + E2:]
    b1r, b2r = b1.reshape(1, -1), b2.reshape(1, -1)

    ea, cpn = _sc_gather(emb_atom, atom_idx, batch_idx, charge_i)
    cpn3 = cpn.reshape(CPAD // BN, 1, BN)
    return _tc_fused(pos_feat, ea, cpn3, W1, b1r, W2, b2r, Wp_a, Wp_h,
                     emb_charge, Wp_c)
